# initial kernel scaffold (unmeasured)
import jax
import jax.numpy as jnp
from jax import lax
from jax.experimental import pallas as pl
from jax.experimental.pallas import tpu as pltpu

N_DEV = 4
K_TILE = 512


def _gelu(y):
    c = 0.7978845608028654
    return 0.5 * y * (1.0 + jnp.tanh(c * (y + 0.044715 * y * y * y)))


def kernel(x, w_mat):
    m_per, k_dim = x.shape
    _, n_dim = w_mat.shape
    n_per = n_dim // N_DEV
    n_k = k_dim // K_TILE
    n_tiles = N_DEV * n_k

    def body(x_ref, w_ref, out_ref, acc_ref, wtile_ref,
             wdma_sems, send_sems, recv_sems, local_sem):
        my = lax.axis_index("i")

        barrier_sem = pltpu.get_barrier_semaphore()
        for off in (1, 2, 3):
            pl.semaphore_signal(
                barrier_sem, inc=1,
                device_id=((my + off) % N_DEV,),
                device_id_type=pl.DeviceIdType.MESH,
            )
        pl.semaphore_wait(barrier_sem, N_DEV - 1)

        def w_dma(t, slot):
            d = (my + 1 + (t // n_k)) % N_DEV
            k = t % n_k
            return pltpu.make_async_copy(
                w_ref.at[pl.ds(k * K_TILE, K_TILE), pl.ds(d * n_per, n_per)],
                wtile_ref.at[slot],
                wdma_sems.at[slot],
            )

        w_dma(0, 0).start()

        rdmas = []
        local_cp = None
        for s in range(N_DEV):
            d = (my + 1 + s) % N_DEV
            slot = s % 2
            if s >= 2:
                rdmas[s - 2].wait_send()
            for k in range(n_k):
                t = s * n_k + k
                wslot = t % 2
                if t + 1 < n_tiles:
                    w_dma(t + 1, (t + 1) % 2).start()
                w_dma(t, wslot).wait()
                xb = x_ref[:, k * K_TILE:(k + 1) * K_TILE].astype(jnp.bfloat16)
                wb = wtile_ref[wslot].astype(jnp.bfloat16)
                p = jnp.dot(xb, wb, preferred_element_type=jnp.float32)
                if k == 0:
                    acc_ref[slot] = p
                else:
                    acc_ref[slot] += p
            acc_ref[slot] = _gelu(acc_ref[slot])
            if s < N_DEV - 1:
                rdma = pltpu.make_async_remote_copy(
                    src_ref=acc_ref.at[slot],
                    dst_ref=out_ref.at[pl.ds(my * m_per, m_per), :],
                    send_sem=send_sems.at[s],
                    recv_sem=recv_sems.at[my],
                    device_id=(d,),
                    device_id_type=pl.DeviceIdType.MESH,
                )
                rdma.start()
                rdmas.append(rdma)
            else:
                local_cp = pltpu.make_async_copy(
                    acc_ref.at[slot],
                    out_ref.at[pl.ds(my * m_per, m_per), :],
                    local_sem,
                )
                local_cp.start()

        for off in (1, 2, 3):
            src = (my + off) % N_DEV
            pltpu.make_async_remote_copy(
                src_ref=acc_ref.at[0],
                dst_ref=out_ref.at[pl.ds(src * m_per, m_per), :],
                send_sem=send_sems.at[N_DEV - 1],
                recv_sem=recv_sems.at[src],
                device_id=(src,),
                device_id_type=pl.DeviceIdType.MESH,
            ).wait_recv()

        rdmas[2].wait_send()
        local_cp.wait()

    return pl.pallas_call(
        body,
        out_shape=jax.ShapeDtypeStruct((N_DEV * m_per, n_per), jnp.float32),
        in_specs=[
            pl.BlockSpec(memory_space=pltpu.VMEM),
            pl.BlockSpec(memory_space=pltpu.ANY),
        ],
        out_specs=pl.BlockSpec(memory_space=pltpu.ANY),
        scratch_shapes=[
            pltpu.VMEM((2, m_per, n_per), jnp.float32),
            pltpu.VMEM((2, K_TILE, n_per), jnp.float32),
            pltpu.SemaphoreType.DMA((2,)),
            pltpu.SemaphoreType.DMA((N_DEV,)),
            pltpu.SemaphoreType.DMA((N_DEV,)),
            pltpu.SemaphoreType.DMA,
        ],
        compiler_params=pltpu.CompilerParams(collective_id=0),
    )(x, w_mat)


# baseline (device time: 261534 ns/iter reference)
import jax
import jax.numpy as jnp
from jax import lax
from jax.experimental import pallas as pl
from jax.experimental.pallas import tpu as pltpu

N_DEV = 4
K_TILE = 512


def _gelu(y):
    c = 0.7978845608028654
    return 0.5 * y * (1.0 + jnp.tanh(c * (y + 0.044715 * y * y * y)))


def kernel(x, w_mat):
    m_per, k_dim = x.shape
    _, n_dim = w_mat.shape
    n_per = n_dim // N_DEV
    n_k = k_dim // K_TILE
    n_tiles = N_DEV * n_k

    def body(x_ref, w_ref, out_ref, acc_ref, wtile_ref,
             wdma_sems, send_sems, recv_sems, local_sem):
        my = lax.axis_index("i")

        barrier_sem = pltpu.get_barrier_semaphore()
        for off in (1, 2, 3):
            pl.semaphore_signal(
                barrier_sem, inc=1,
                device_id=((my + off) % N_DEV,),
                device_id_type=pl.DeviceIdType.MESH,
            )
        pl.semaphore_wait(barrier_sem, N_DEV - 1)

        def w_dma(t, slot):
            d = (my + 1 + (t // n_k)) % N_DEV
            k = t % n_k
            return pltpu.make_async_copy(
                w_ref.at[pl.ds(k * K_TILE, K_TILE), pl.ds(d * n_per, n_per)],
                wtile_ref.at[slot],
                wdma_sems.at[slot],
            )

        w_dma(0, 0).start()

        rdmas = []
        local_cp = None
        for s in range(N_DEV):
            d = (my + 1 + s) % N_DEV
            slot = s % 2
            if s >= 2:
                rdmas[s - 2].wait_send()
            for k in range(n_k):
                t = s * n_k + k
                wslot = t % 2
                if t + 1 < n_tiles:
                    w_dma(t + 1, (t + 1) % 2).start()
                w_dma(t, wslot).wait()
                xb = x_ref[:, k * K_TILE:(k + 1) * K_TILE].astype(jnp.bfloat16)
                wb = wtile_ref[wslot].astype(jnp.bfloat16)
                p = jnp.dot(xb, wb, preferred_element_type=jnp.float32)
                if k == 0:
                    acc_ref[slot] = p
                else:
                    acc_ref[slot] += p
            acc_ref[slot] = _gelu(acc_ref[slot])
            if s < N_DEV - 1:
                rdma = pltpu.make_async_remote_copy(
                    src_ref=acc_ref.at[slot],
                    dst_ref=out_ref.at[pl.ds(my * m_per, m_per), :],
                    send_sem=send_sems.at[s],
                    recv_sem=recv_sems.at[my],
                    device_id=(d,),
                    device_id_type=pl.DeviceIdType.MESH,
                )
                rdma.start()
                rdmas.append(rdma)
            else:
                local_cp = pltpu.make_async_copy(
                    acc_ref.at[slot],
                    out_ref.at[pl.ds(my * m_per, m_per), :],
                    local_sem,
                )
                local_cp.start()

        for off in (1, 2, 3):
            src = (my + off) % N_DEV
            pltpu.make_async_remote_copy(
                src_ref=acc_ref.at[0],
                dst_ref=out_ref.at[pl.ds(src * m_per, m_per), :],
                send_sem=send_sems.at[N_DEV - 1],
                recv_sem=recv_sems.at[src],
                device_id=(src,),
                device_id_type=pl.DeviceIdType.MESH,
            ).wait_recv()

        rdmas[2].wait_send()
        local_cp.wait()

    return pl.pallas_call(
        body,
        out_shape=jax.ShapeDtypeStruct((N_DEV * m_per, n_per), jnp.float32),
        in_specs=[
            pl.BlockSpec(memory_space=pltpu.VMEM),
            pl.BlockSpec(memory_space=pltpu.HBM),
        ],
        out_specs=pl.BlockSpec(memory_space=pltpu.HBM),
        scratch_shapes=[
            pltpu.VMEM((2, m_per, n_per), jnp.float32),
            pltpu.VMEM((2, K_TILE, n_per), jnp.float32),
            pltpu.SemaphoreType.DMA((2,)),
            pltpu.SemaphoreType.DMA((N_DEV,)),
            pltpu.SemaphoreType.DMA((N_DEV,)),
            pltpu.SemaphoreType.DMA,
        ],
        compiler_params=pltpu.CompilerParams(collective_id=0),
    )(x, w_mat)


# device time: 160585 ns/iter; 1.6286x vs baseline; 1.6286x over previous
import jax
import jax.numpy as jnp
from jax import lax
from jax.experimental import pallas as pl
from jax.experimental.pallas import tpu as pltpu

N_DEV = 4
K_TILE = 512
DEST_OFFSETS = (2, 1, 3, 0)


def _gelu(y):
    c = 0.7978845608028654
    return 0.5 * y * (1.0 + jnp.tanh(c * (y + 0.044715 * y * y * y)))


def kernel(x, w_mat):
    m_per, k_dim = x.shape
    _, n_dim = w_mat.shape
    n_per = n_dim // N_DEV
    n_k = k_dim // K_TILE
    n_tiles = N_DEV * n_k

    def body(x_ref, w_ref, out_ref, acc_ref, wtile_ref, send_ref, recv_ref,
             wdma_sems, send_sems, recv_sems, out_sem):
        my = lax.axis_index("i")

        barrier_sem = pltpu.get_barrier_semaphore()
        for off in (1, 2, 3):
            pl.semaphore_signal(
                barrier_sem, inc=1,
                device_id=((my + off) % N_DEV,),
                device_id_type=pl.DeviceIdType.MESH,
            )
        pl.semaphore_wait(barrier_sem, N_DEV - 1)

        def w_dma(t, slot):
            d = (my + DEST_OFFSETS[t // n_k]) % N_DEV
            k = t % n_k
            return pltpu.make_async_copy(
                w_ref.at[pl.ds(k * K_TILE, K_TILE), pl.ds(d * n_per, n_per)],
                wtile_ref.at[slot],
                wdma_sems.at[slot],
            )

        w_dma(0, 0).start()

        rdmas = []
        for s, off in enumerate(DEST_OFFSETS):
            d = (my + off) % N_DEV
            for k in range(n_k):
                t = s * n_k + k
                wslot = t % 2
                if t + 1 < n_tiles:
                    w_dma(t + 1, (t + 1) % 2).start()
                w_dma(t, wslot).wait()
                xb = x_ref[:, k * K_TILE:(k + 1) * K_TILE].astype(jnp.bfloat16)
                wb = wtile_ref[wslot].astype(jnp.bfloat16)
                p = jnp.dot(xb, wb, preferred_element_type=jnp.float32)
                if k == 0:
                    acc_ref[...] = p
                else:
                    acc_ref[...] += p
            acc_ref[...] = _gelu(acc_ref[...])
            if off != 0:
                sslot = s % 2
                if s >= 2:
                    rdmas[s - 2].wait_send()
                send_ref[sslot] = acc_ref[...].astype(jnp.bfloat16)
                rslot = (my - d - 1) % N_DEV
                rdma = pltpu.make_async_remote_copy(
                    src_ref=send_ref.at[sslot],
                    dst_ref=recv_ref.at[rslot],
                    send_sem=send_sems.at[s],
                    recv_sem=recv_sems.at[rslot],
                    device_id=(d,),
                    device_id_type=pl.DeviceIdType.MESH,
                )
                rdma.start()
                rdmas.append(rdma)
            else:
                own_cp = pltpu.make_async_copy(
                    acc_ref, out_ref.at[pl.ds(my * m_per, m_per), :], out_sem)
                own_cp.start()
                own_cp.wait()

        for j in range(N_DEV - 1):
            src = (my + 1 + j) % N_DEV
            pltpu.make_async_remote_copy(
                src_ref=send_ref.at[0],
                dst_ref=recv_ref.at[j],
                send_sem=send_sems.at[N_DEV - 1],
                recv_sem=recv_sems.at[j],
                device_id=(src,),
                device_id_type=pl.DeviceIdType.MESH,
            ).wait_recv()
            acc_ref[...] = recv_ref[j].astype(jnp.float32)
            cp = pltpu.make_async_copy(
                acc_ref, out_ref.at[pl.ds(src * m_per, m_per), :], out_sem)
            cp.start()
            cp.wait()

        rdmas[1].wait_send()
        rdmas[2].wait_send()

    return pl.pallas_call(
        body,
        out_shape=jax.ShapeDtypeStruct((N_DEV * m_per, n_per), jnp.float32),
        in_specs=[
            pl.BlockSpec(memory_space=pltpu.VMEM),
            pl.BlockSpec(memory_space=pltpu.HBM),
        ],
        out_specs=pl.BlockSpec(memory_space=pltpu.HBM),
        scratch_shapes=[
            pltpu.VMEM((m_per, n_per), jnp.float32),
            pltpu.VMEM((2, K_TILE, n_per), jnp.float32),
            pltpu.VMEM((2, m_per, n_per), jnp.bfloat16),
            pltpu.VMEM((3, m_per, n_per), jnp.bfloat16),
            pltpu.SemaphoreType.DMA((2,)),
            pltpu.SemaphoreType.DMA((N_DEV,)),
            pltpu.SemaphoreType.DMA((N_DEV,)),
            pltpu.SemaphoreType.DMA,
        ],
        compiler_params=pltpu.CompilerParams(
            collective_id=0,
            vmem_limit_bytes=64 * 1024 * 1024,
        ),
    )(x, w_mat)


# device time: 146596 ns/iter; 1.7840x vs baseline; 1.0954x over previous
import jax
import jax.numpy as jnp
from jax import lax
from jax.experimental import pallas as pl
from jax.experimental.pallas import tpu as pltpu

N_DEV = 4
K_CHUNK = 1024
X_TILE = 512
HALF = 1024
DEST_OFFSETS = (2, 1, 3, 0)
N_JOBS = 2 * N_DEV


def _gelu(y):
    c = 0.7978845608028654
    return 0.5 * y * (1.0 + jnp.tanh(c * (y + 0.044715 * y * y * y)))


def kernel(x, w_mat):
    m_per, k_dim = x.shape
    _, n_dim = w_mat.shape
    n_per = n_dim // N_DEV
    n_c = k_dim // K_CHUNK
    n_xt = k_dim // X_TILE
    n_tiles = N_JOBS * n_c

    def body(x_ref, w_ref, out_ref, xstage_ref, xbf_ref, wtile_ref,
             acc_ref, stage_ref, send_ref, recv_ref,
             xdma_sems, wdma_sems, send_sems, recv_sems, out_sems):
        my = lax.axis_index("i")

        barrier_sem = pltpu.get_barrier_semaphore()
        for off in (1, 2, 3):
            pl.semaphore_signal(
                barrier_sem, inc=1,
                device_id=((my + off) % N_DEV,),
                device_id_type=pl.DeviceIdType.MESH,
            )

        def x_dma(i, slot):
            return pltpu.make_async_copy(
                x_ref.at[:, pl.ds(i * X_TILE, X_TILE)],
                xstage_ref.at[slot],
                xdma_sems.at[slot],
            )

        def w_dma(t, slot):
            jj, c = t // n_c, t % n_c
            d = (my + DEST_OFFSETS[jj // 2]) % N_DEV
            col = d * n_per + (jj % 2) * HALF
            return pltpu.make_async_copy(
                w_ref.at[pl.ds(c * K_CHUNK, K_CHUNK), pl.ds(col, HALF)],
                wtile_ref.at[slot],
                wdma_sems.at[slot],
            )

        x_dma(0, 0).start()
        w_dma(0, 0).start()
        w_dma(1, 1).start()
        for i in range(n_xt):
            if i + 1 < n_xt:
                x_dma(i + 1, (i + 1) % 2).start()
            x_dma(i, i % 2).wait()
            xbf_ref[:, i * X_TILE:(i + 1) * X_TILE] = (
                xstage_ref[i % 2].astype(jnp.bfloat16))

        rdmas = []
        own_cps = []

        def compute_job(jj):
            h = jj % 2
            d = (my + DEST_OFFSETS[jj // 2]) % N_DEV
            for c in range(n_c):
                t = jj * n_c + c
                w_dma(t, t % 2).wait()
                p = jnp.dot(
                    xbf_ref[:, c * K_CHUNK:(c + 1) * K_CHUNK],
                    wtile_ref[t % 2].astype(jnp.bfloat16),
                    preferred_element_type=jnp.float32)
                if c == 0:
                    acc_ref[...] = p
                else:
                    acc_ref[...] += p
                if t + 2 < n_tiles:
                    w_dma(t + 2, t % 2).start()
            acc_ref[...] = _gelu(acc_ref[...])
            if jj < 6:
                sslot = jj % 4
                if jj >= 4:
                    rdmas[jj - 4].wait_send()
                send_ref[sslot] = acc_ref[...].astype(jnp.bfloat16)
                if jj == 0:
                    pl.semaphore_wait(barrier_sem, N_DEV - 1)
                rslot = (my - d - 1) % N_DEV
                rdma = pltpu.make_async_remote_copy(
                    src_ref=send_ref.at[sslot],
                    dst_ref=recv_ref.at[rslot, :, pl.ds(h * HALF, HALF)],
                    send_sem=send_sems.at[jj],
                    recv_sem=recv_sems.at[2 * rslot + h],
                    device_id=(d,),
                    device_id_type=pl.DeviceIdType.MESH,
                )
                rdma.start()
                rdmas.append(rdma)
            else:
                cp = pltpu.make_async_copy(
                    acc_ref,
                    out_ref.at[pl.ds(my * m_per, m_per), pl.ds(h * HALF, HALF)],
                    out_sems.at[h])
                cp.start()
                own_cps.append(cp)

        def drain(j, last_cp):
            src = (my + 1 + j) % N_DEV
            for h in range(2):
                pltpu.make_async_remote_copy(
                    src_ref=send_ref.at[0],
                    dst_ref=recv_ref.at[j, :, pl.ds(h * HALF, HALF)],
                    send_sem=send_sems.at[6],
                    recv_sem=recv_sems.at[2 * j + h],
                    device_id=(src,),
                    device_id_type=pl.DeviceIdType.MESH,
                ).wait_recv()
            if last_cp is not None:
                last_cp.wait()
            stage_ref[...] = recv_ref[j].astype(jnp.float32)
            cp = pltpu.make_async_copy(
                stage_ref, out_ref.at[pl.ds(src * m_per, m_per), :],
                out_sems.at[2])
            cp.start()
            return cp

        for jj in range(6):
            compute_job(jj)
        compute_job(6)
        cp1 = drain(1, None)
        own_cps[0].wait()
        compute_job(7)
        cp0 = drain(0, cp1)
        cp2 = drain(2, cp0)

        for jj in (2, 3, 4, 5):
            rdmas[jj].wait_send()
        own_cps[1].wait()
        cp2.wait()

    return pl.pallas_call(
        body,
        out_shape=jax.ShapeDtypeStruct((N_DEV * m_per, n_per), jnp.float32),
        in_specs=[
            pl.BlockSpec(memory_space=pltpu.HBM),
            pl.BlockSpec(memory_space=pltpu.HBM),
        ],
        out_specs=pl.BlockSpec(memory_space=pltpu.HBM),
        scratch_shapes=[
            pltpu.VMEM((2, m_per, X_TILE), jnp.float32),
            pltpu.VMEM((m_per, k_dim), jnp.bfloat16),
            pltpu.VMEM((2, K_CHUNK, HALF), jnp.float32),
            pltpu.VMEM((m_per, HALF), jnp.float32),
            pltpu.VMEM((m_per, n_per), jnp.float32),
            pltpu.VMEM((4, m_per, HALF), jnp.bfloat16),
            pltpu.VMEM((3, m_per, n_per), jnp.bfloat16),
            pltpu.SemaphoreType.DMA((2,)),
            pltpu.SemaphoreType.DMA((2,)),
            pltpu.SemaphoreType.DMA((7,)),
            pltpu.SemaphoreType.DMA((6,)),
            pltpu.SemaphoreType.DMA((3,)),
        ],
        compiler_params=pltpu.CompilerParams(
            collective_id=0,
            vmem_limit_bytes=64 * 1024 * 1024,
        ),
    )(x, w_mat)
